# baseline (speedup 1.0000x reference)
import dataclasses
import functools

import numpy as np
import jax
import jax.numpy as jnp
from jax import lax
from jax.experimental import pallas as pl
from jax.experimental.pallas import tpu as pltpu
from jax.experimental.pallas import tpu_sc as plsc

NF = 26
D = 16
P = NF * (NF - 1) // 2
ROWW = NF * D
GRP = 16
NC, NS = 2, 16
NW = NC * NS

_IU, _JU = np.triu_indices(NF, k=1)


def _compute_row(e_v, xfcol, vbc, ebase):
    acc = jnp.zeros((D,), jnp.float32)
    p = 0
    for i in range(NF - 1):
        acc_i = jnp.zeros((D,), jnp.float32)
        for j in range(i + 1, NF):
            vi = e_v[ebase + i, pl.ds(xfcol[j], D)]
            vj = e_v[ebase + j, pl.ds(xfcol[i], D)]
            acc_i = acc_i + vi * vj * vbc[j]
            p += 1
        acc = acc + acc_i * vbc[i]
    return acc


def _ffm_body(rows_per_w, emb_hbm, xpair_hbm, xf_hbm, val_hbm, out_hbm,
              xpair_v, xf_v, val_v, e_a, e_b, e_c, e_d, acc_v, out_v,
              sem_a, sem_b, sem_c, sem_d):
    core = lax.axis_index("c")
    sub = lax.axis_index("s")
    wid = sub * NC + core
    base = wid * rows_per_w
    npairs = rows_per_w // 2
    acc_v[...] = jnp.zeros((D,), jnp.float32)

    pltpu.sync_copy(xpair_hbm.at[pl.ds(base, rows_per_w)], xpair_v)
    pltpu.sync_copy(xf_hbm.at[pl.ds(base * 32, rows_per_w * 32)], xf_v)
    pltpu.sync_copy(val_hbm.at[pl.ds(base * 32, rows_per_w * 32)], val_v)

    def finish_row(r, acc):
        s = jnp.sum(acc)
        lane = lax.rem(r, D)
        sel = lax.iota(jnp.int32, D) == lane
        acc_v[...] = acc_v[...] + jnp.where(sel, s, jnp.float32(0.0))

        @pl.when(lane == D - 1)
        def _flush():
            off = pl.multiple_of((r // D) * D, D)
            out_v[pl.ds(off, D)] = acc_v[...]
            acc_v[...] = jnp.zeros((D,), jnp.float32)

    def xfcols(r):
        o = r * 32
        a = xf_v[pl.ds(o, D)]
        b = xf_v[pl.ds(o + D, D)]
        return [(a[j] if j < D else b[j - D]) * D for j in range(NF)]

    def vbcs(r):
        o = r * 32
        a = val_v[pl.ds(o, D)]
        b = val_v[pl.ds(o + D, D)]
        return [jnp.full((D,), a[j] if j < D else b[j - D], jnp.float32)
                for j in range(NF)]

    def compute_one(e_v, r):
        acc0 = _compute_row(e_v, xfcols(r), vbcs(r), 0)
        finish_row(r, acc0)

    def issue(row_idx, e_v, sem):
        p = jnp.minimum(row_idx, rows_per_w - 1)
        pltpu.async_copy(emb_hbm.at[xpair_v.at[p]], e_v, sem)

    def drain(e_v, sem):
        pltpu.make_async_copy(emb_hbm.at[xpair_v.at[0]], e_v, sem).wait()

    bufs = (e_a, e_b, e_c, e_d)
    sems = (sem_a, sem_b, sem_c, sem_d)
    for k in range(3):
        issue(k, bufs[k], sems[k])

    @pl.loop(0, rows_per_w, step=4)
    def _rows(r):
        for k in range(4):
            drain(bufs[k], sems[k])
            issue(r + k + 3, bufs[(k + 3) % 4], sems[(k + 3) % 4])
            compute_one(bufs[k], r + k)

    for k in range(3):
        drain(bufs[k], sems[k])
    pltpu.sync_copy(out_v, out_hbm.at[pl.ds(base, rows_per_w)])


def kernel(x_field, x, x_val, emb):
    batch = x.shape[0]
    total = emb.shape[1]
    rows_per_w = batch // NW

    emb_t = jnp.transpose(emb, (1, 0, 2)).reshape(total, ROWW)
    xpair = x.astype(jnp.int32)
    xf_flat = jnp.pad(x_field.astype(jnp.int32),
                      ((0, 0), (0, 32 - NF))).reshape(batch * 32)
    val = jnp.pad(x_val.astype(jnp.float32),
                  ((0, 0), (0, 32 - NF))).reshape(batch * 32)

    mesh = plsc.VectorSubcoreMesh(core_axis_name="c", subcore_axis_name="s",
                                  num_cores=NC, num_subcores=NS)
    cp = pltpu.CompilerParams()
    if "needs_layout_passes" in pltpu.CompilerParams.__dataclass_fields__:
        cp = dataclasses.replace(cp, needs_layout_passes=False)
    if "use_tc_tiling_on_sc" in pltpu.CompilerParams.__dataclass_fields__:
        cp = dataclasses.replace(cp, use_tc_tiling_on_sc=False)
    k = pl.kernel(
        functools.partial(_ffm_body, rows_per_w),
        out_type=jax.ShapeDtypeStruct((batch,), jnp.float32),
        mesh=mesh,
        scratch_types=[
            pltpu.VMEM((rows_per_w, NF), jnp.int32),
            pltpu.VMEM((rows_per_w * 32,), jnp.int32),
            pltpu.VMEM((rows_per_w * 32,), jnp.float32),
            pltpu.VMEM((NF, ROWW), jnp.float32),
            pltpu.VMEM((NF, ROWW), jnp.float32),
            pltpu.VMEM((NF, ROWW), jnp.float32),
            pltpu.VMEM((NF, ROWW), jnp.float32),
            pltpu.VMEM((D,), jnp.float32),
            pltpu.VMEM((rows_per_w,), jnp.float32),
            pltpu.SemaphoreType.DMA,
            pltpu.SemaphoreType.DMA,
            pltpu.SemaphoreType.DMA,
            pltpu.SemaphoreType.DMA,
        ],
        compiler_params=cp,
    )
    return k(emb_t, xpair, xf_flat, val)


# speedup vs baseline: 1.0123x; 1.0123x over previous
import dataclasses
import functools

import numpy as np
import jax
import jax.numpy as jnp
from jax import lax
from jax.experimental import pallas as pl
from jax.experimental.pallas import tpu as pltpu
from jax.experimental.pallas import tpu_sc as plsc

NF = 26
D = 16
P = NF * (NF - 1) // 2
ROWW = NF * D
GRP = 16
NC, NS = 2, 16
NW = NC * NS

_IU, _JU = np.triu_indices(NF, k=1)


def _compute_row(e_v, xfcol, vbc, ebase):
    acc = jnp.zeros((D,), jnp.float32)
    p = 0
    for i in range(NF - 1):
        acc_i = jnp.zeros((D,), jnp.float32)
        for j in range(i + 1, NF):
            vi = e_v[ebase + i, pl.ds(xfcol[j], D)]
            vj = e_v[ebase + j, pl.ds(xfcol[i], D)]
            acc_i = acc_i + vi * vj * vbc[j]
            p += 1
        acc = acc + acc_i * vbc[i]
    return acc


def _ffm_body(rows_per_w, emb_hbm, xpair_hbm, xf_hbm, val_hbm, out_hbm,
              xpair_v, xf_v, val_v, e_a, e_b, acc_v, out_v, sem_a, sem_b):
    core = lax.axis_index("c")
    sub = lax.axis_index("s")
    wid = sub * NC + core
    base = wid * rows_per_w
    npairs = rows_per_w // 2
    acc_v[...] = jnp.zeros((D,), jnp.float32)

    pltpu.sync_copy(xpair_hbm.at[pl.ds(base // 2, npairs)], xpair_v)
    pltpu.sync_copy(xf_hbm.at[pl.ds(base * 32, rows_per_w * 32)], xf_v)
    pltpu.sync_copy(val_hbm.at[pl.ds(base * 32, rows_per_w * 32)], val_v)

    def finish_row(r, acc):
        s = jnp.sum(acc)
        lane = lax.rem(r, D)
        sel = lax.iota(jnp.int32, D) == lane
        acc_v[...] = acc_v[...] + jnp.where(sel, s, jnp.float32(0.0))

        @pl.when(lane == D - 1)
        def _flush():
            off = pl.multiple_of((r // D) * D, D)
            out_v[pl.ds(off, D)] = acc_v[...]
            acc_v[...] = jnp.zeros((D,), jnp.float32)

    def xfcols(r):
        o = r * 32
        a = xf_v[pl.ds(o, D)]
        b = xf_v[pl.ds(o + D, D)]
        return [(a[j] if j < D else b[j - D]) * D for j in range(NF)]

    def vbcs(r):
        o = r * 32
        a = val_v[pl.ds(o, D)]
        b = val_v[pl.ds(o + D, D)]
        return [jnp.full((D,), a[j] if j < D else b[j - D], jnp.float32)
                for j in range(NF)]

    def compute_pair(e_v, r):
        acc0 = _compute_row(e_v, xfcols(r), vbcs(r), 0)
        finish_row(r, acc0)
        acc1 = _compute_row(e_v, xfcols(r + 1), vbcs(r + 1), NF)
        finish_row(r + 1, acc1)

    def issue(pair_idx, e_v, sem):
        p = jnp.minimum(pair_idx, npairs - 1)
        pltpu.async_copy(emb_hbm.at[xpair_v.at[p]], e_v, sem)

    def drain(e_v, sem):
        pltpu.make_async_copy(emb_hbm.at[xpair_v.at[0]], e_v, sem).wait()

    issue(0, e_a, sem_a)

    @pl.loop(0, rows_per_w, step=4)
    def _rows(r):
        q = r // 2
        issue(q + 1, e_b, sem_b)
        drain(e_a, sem_a)
        compute_pair(e_a, r)
        issue(q + 2, e_a, sem_a)
        drain(e_b, sem_b)
        compute_pair(e_b, r + 2)

    drain(e_a, sem_a)
    pltpu.sync_copy(out_v, out_hbm.at[pl.ds(base, rows_per_w)])


def kernel(x_field, x, x_val, emb):
    batch = x.shape[0]
    total = emb.shape[1]
    rows_per_w = batch // NW

    emb_t = jnp.transpose(emb, (1, 0, 2)).reshape(total, ROWW)
    xpair = x.astype(jnp.int32).reshape(batch // 2, 2 * NF)
    xf_flat = jnp.pad(x_field.astype(jnp.int32),
                      ((0, 0), (0, 32 - NF))).reshape(batch * 32)
    val = jnp.pad(x_val.astype(jnp.float32),
                  ((0, 0), (0, 32 - NF))).reshape(batch * 32)

    mesh = plsc.VectorSubcoreMesh(core_axis_name="c", subcore_axis_name="s",
                                  num_cores=NC, num_subcores=NS)
    cp = pltpu.CompilerParams()
    if "needs_layout_passes" in pltpu.CompilerParams.__dataclass_fields__:
        cp = dataclasses.replace(cp, needs_layout_passes=False)
    if "use_tc_tiling_on_sc" in pltpu.CompilerParams.__dataclass_fields__:
        cp = dataclasses.replace(cp, use_tc_tiling_on_sc=False)
    k = pl.kernel(
        functools.partial(_ffm_body, rows_per_w),
        out_type=jax.ShapeDtypeStruct((batch,), jnp.float32),
        mesh=mesh,
        scratch_types=[
            pltpu.VMEM((rows_per_w // 2, 2 * NF), jnp.int32),
            pltpu.VMEM((rows_per_w * 32,), jnp.int32),
            pltpu.VMEM((rows_per_w * 32,), jnp.float32),
            pltpu.VMEM((2 * NF, ROWW), jnp.float32),
            pltpu.VMEM((2 * NF, ROWW), jnp.float32),
            pltpu.VMEM((D,), jnp.float32),
            pltpu.VMEM((rows_per_w,), jnp.float32),
            pltpu.SemaphoreType.DMA,
            pltpu.SemaphoreType.DMA,
        ],
        compiler_params=cp,
    )
    return k(emb_t, xpair, xf_flat, val)
